# baseline (speedup 1.0000x reference)
import functools

import jax
import jax.numpy as jnp
from jax import lax
from jax.experimental import pallas as pl
from jax.experimental.pallas import tpu as pltpu
from jax.experimental.pallas import tpu_sc as plsc

_CHUNK = 2688
_FULL_CHUNKS = 372
_PER_WORKER = _FULL_CHUNKS // 4
_TAIL_OFF = _FULL_CHUNKS * _CHUNK
_TAIL = 1000000 - _TAIL_OFF
_NBUF = 6


def _sc_copy_body(src, dst, buf, tail_buf, in_sems, out_sems):
    c = lax.axis_index("c")
    s = lax.axis_index("s")
    wid = s * 2 + c
    band = wid // 4
    q = wid % 4
    rows = pl.ds(band * 8, 8)

    def cols(j):
        return pl.ds((q + 4 * j) * _CHUNK, _CHUNK)

    def step(j, _):
        b = j % _NBUF

        @pl.when(j < _PER_WORKER)
        def _():
            @pl.when(j >= _NBUF)
            def _():
                pltpu.make_async_copy(buf.at[b], dst.at[rows, cols(j - _NBUF)],
                                      out_sems.at[b]).wait()

            pltpu.make_async_copy(src.at[rows, cols(j)], buf.at[b],
                                  in_sems.at[b]).start()

        @pl.when(j >= 1)
        def _():
            b1 = (j - 1) % _NBUF
            pltpu.make_async_copy(src.at[rows, cols(j - 1)], buf.at[b1],
                                  in_sems.at[b1]).wait()
            pltpu.make_async_copy(buf.at[b1], dst.at[rows, cols(j - 1)],
                                  out_sems.at[b1]).start()

        return _

    lax.fori_loop(0, _PER_WORKER + 1, step, None)

    def drain(j, _):
        b = j % _NBUF
        pltpu.make_async_copy(buf.at[b], dst.at[rows, cols(j)],
                              out_sems.at[b]).wait()
        return _

    lax.fori_loop(_PER_WORKER - _NBUF, _PER_WORKER, drain, None)

    @pl.when(q == 0)
    def _():
        tcols = pl.ds(_TAIL_OFF, _TAIL)
        pltpu.sync_copy(src.at[rows, tcols], tail_buf)
        pltpu.sync_copy(tail_buf, dst.at[rows, tcols])


def kernel(inputs, p_emb):
    n, d = p_emb.shape
    t = p_emb.T
    mesh = plsc.VectorSubcoreMesh(core_axis_name="c", subcore_axis_name="s")
    sc_copy = functools.partial(
        pl.kernel,
        mesh=mesh,
        out_type=jax.ShapeDtypeStruct(t.shape, t.dtype),
        scratch_types=[
            pltpu.VMEM((_NBUF, 8, _CHUNK), jnp.float32),
            pltpu.VMEM((8, _TAIL), jnp.float32),
            pltpu.SemaphoreType.DMA((_NBUF,)),
            pltpu.SemaphoreType.DMA((_NBUF,)),
        ],
    )(_sc_copy_body)
    return sc_copy(t).T


# speedup vs baseline: 1.0669x; 1.0669x over previous
import functools

import jax
import jax.numpy as jnp
from jax import lax
from jax.experimental import pallas as pl
from jax.experimental.pallas import tpu as pltpu
from jax.experimental.pallas import tpu_sc as plsc

_CHUNK = 2688
_FULL_CHUNKS = 372
_PER_WORKER = _FULL_CHUNKS // 4
_TAIL_OFF = _FULL_CHUNKS * _CHUNK
_TAIL = 1000000 - _TAIL_OFF
_NBUF = 2


def _sc_copy_body(src, dst, shared, tail_buf, in_sems, out_sems):
    c = lax.axis_index("c")
    s = lax.axis_index("s")
    wid = s * 2 + c
    band = wid // 4
    q = wid % 4
    rows = pl.ds(band * 8, 8)

    def cols(j):
        return pl.ds((q + 4 * j) * _CHUNK, _CHUNK)

    def buf(b):
        return shared.at[s, b]

    def step(j, _):
        b = j % _NBUF

        @pl.when(j < _PER_WORKER)
        def _():
            @pl.when(j >= _NBUF)
            def _():
                pltpu.make_async_copy(buf(b), dst.at[rows, cols(j - _NBUF)],
                                      out_sems.at[b]).wait()

            pltpu.make_async_copy(src.at[rows, cols(j)], buf(b),
                                  in_sems.at[b]).start()

        @pl.when(j >= 1)
        def _():
            b1 = (j - 1) % _NBUF
            pltpu.make_async_copy(src.at[rows, cols(j - 1)], buf(b1),
                                  in_sems.at[b1]).wait()
            pltpu.make_async_copy(buf(b1), dst.at[rows, cols(j - 1)],
                                  out_sems.at[b1]).start()

        return _

    lax.fori_loop(0, _PER_WORKER + 1, step, None)

    def drain(j, _):
        b = j % _NBUF
        pltpu.make_async_copy(buf(b), dst.at[rows, cols(j)],
                              out_sems.at[b]).wait()
        return _

    lax.fori_loop(_PER_WORKER - _NBUF, _PER_WORKER, drain, None)

    @pl.when(q == 0)
    def _():
        tcols = pl.ds(_TAIL_OFF, _TAIL)
        pltpu.sync_copy(src.at[rows, tcols], tail_buf)
        pltpu.sync_copy(tail_buf, dst.at[rows, tcols])


def kernel(inputs, p_emb):
    n, d = p_emb.shape
    t = p_emb.T
    mesh = plsc.VectorSubcoreMesh(core_axis_name="c", subcore_axis_name="s")
    sc_copy = functools.partial(
        pl.kernel,
        mesh=mesh,
        out_type=jax.ShapeDtypeStruct(t.shape, t.dtype),
        scratch_types=[
            pltpu.VMEM_SHARED((16, _NBUF, 8, _CHUNK), jnp.float32),
            pltpu.VMEM((8, _TAIL), jnp.float32),
            pltpu.SemaphoreType.DMA((_NBUF,)),
            pltpu.SemaphoreType.DMA((_NBUF,)),
        ],
    )(_sc_copy_body)
    return sc_copy(t).T
